# baseline (speedup 1.0000x reference)
import jax
import jax.numpy as jnp
from jax.experimental import pallas as pl
from jax.experimental.pallas import tpu as pltpu

_BLOCK_M = 4000


def _affine_kernel(t_ref, w_ref, b_ref, o_ref):
    o_ref[...] = (
        jnp.dot(t_ref[...], w_ref[...], preferred_element_type=jnp.float32)
        + b_ref[...]
    )


def kernel(dummy, table, W, b):
    M, D = table.shape
    M2 = M // 2
    D2 = 2 * D
    t2 = table.reshape(M2, D2)
    zeros = jnp.zeros((D, D), W.dtype)
    w2 = jnp.block([[W, zeros], [zeros, W]])
    b2 = jnp.concatenate([b, b]).reshape(1, D2)

    grid = (M2 // _BLOCK_M,)
    out = pl.pallas_call(
        _affine_kernel,
        grid=grid,
        in_specs=[
            pl.BlockSpec((_BLOCK_M, D2), lambda i: (i, 0)),
            pl.BlockSpec((D2, D2), lambda i: (0, 0)),
            pl.BlockSpec((1, D2), lambda i: (0, 0)),
        ],
        out_specs=pl.BlockSpec((_BLOCK_M, D2), lambda i: (i, 0)),
        out_shape=jax.ShapeDtypeStruct((M2, D2), jnp.float32),
        compiler_params=pltpu.CompilerParams(
            dimension_semantics=("arbitrary",),
        ),
    )(t2, w2, b2)
    return out.reshape(M, D)


# speedup vs baseline: 1.3882x; 1.3882x over previous
import jax
import jax.numpy as jnp
from jax.experimental import pallas as pl
from jax.experimental.pallas import tpu as pltpu

_BLOCK_M = 8000


def _affine_kernel(t_ref, w_ref, b_ref, o_ref):
    o_ref[...] = (
        jnp.dot(t_ref[...], w_ref[...], preferred_element_type=jnp.float32)
        + b_ref[...]
    )


def kernel(dummy, table, W, b):
    M, D = table.shape
    b2 = b.reshape(1, D)

    grid = (M // _BLOCK_M,)
    out = pl.pallas_call(
        _affine_kernel,
        grid=grid,
        in_specs=[
            pl.BlockSpec((_BLOCK_M, D), lambda i: (i, 0)),
            pl.BlockSpec((D, D), lambda i: (0, 0)),
            pl.BlockSpec((1, D), lambda i: (0, 0)),
        ],
        out_specs=pl.BlockSpec((_BLOCK_M, D), lambda i: (i, 0)),
        out_shape=jax.ShapeDtypeStruct((M, D), jnp.float32),
        compiler_params=pltpu.CompilerParams(
            dimension_semantics=("arbitrary",),
        ),
    )(table, W, b2)
    return out


# speedup vs baseline: 1.3939x; 1.0041x over previous
import jax
import jax.numpy as jnp
from jax.experimental import pallas as pl
from jax.experimental.pallas import tpu as pltpu

_BLOCK_M = 20000


def _affine_kernel(t_ref, w_ref, b_ref, o_ref):
    o_ref[...] = (
        jnp.dot(t_ref[...], w_ref[...], preferred_element_type=jnp.float32)
        + b_ref[...]
    )


def kernel(dummy, table, W, b):
    M, D = table.shape
    b2 = b.reshape(1, D)

    grid = (M // _BLOCK_M,)
    out = pl.pallas_call(
        _affine_kernel,
        grid=grid,
        in_specs=[
            pl.BlockSpec((_BLOCK_M, D), lambda i: (i, 0)),
            pl.BlockSpec((D, D), lambda i: (0, 0)),
            pl.BlockSpec((1, D), lambda i: (0, 0)),
        ],
        out_specs=pl.BlockSpec((_BLOCK_M, D), lambda i: (i, 0)),
        out_shape=jax.ShapeDtypeStruct((M, D), jnp.float32),
        compiler_params=pltpu.CompilerParams(
            dimension_semantics=("arbitrary",),
        ),
    )(table, W, b2)
    return out


# speedup vs baseline: 8.5883x; 6.1613x over previous
import jax
import jax.numpy as jnp
from jax.experimental import pallas as pl
from jax.experimental.pallas import tpu as pltpu

_CHUNK = 31232
_N_FULL = 32
_TAIL = 1_000_000 - _N_FULL * _CHUNK


def _affine_kernel(wT_ref, b_ref, tT_hbm, outT_hbm, in_buf, out_buf,
                   tail_in, tail_out, in_sems, out_sems, tail_sems):
    i = pl.program_id(0)
    slot = jax.lax.rem(i, 2)
    nxt_slot = jax.lax.rem(i + 1, 2)

    def in_copy(chunk_idx, buf_slot):
        return pltpu.make_async_copy(
            tT_hbm.at[:, pl.ds(chunk_idx * _CHUNK, _CHUNK)],
            in_buf.at[buf_slot],
            in_sems.at[buf_slot],
        )

    def out_copy(buf_slot, chunk_idx):
        return pltpu.make_async_copy(
            out_buf.at[buf_slot],
            outT_hbm.at[:, pl.ds(chunk_idx * _CHUNK, _CHUNK)],
            out_sems.at[buf_slot],
        )

    def tail_in_copy():
        return pltpu.make_async_copy(
            tT_hbm.at[:, pl.ds(_N_FULL * _CHUNK, _TAIL)],
            tail_in,
            tail_sems.at[0],
        )

    def tail_out_copy():
        return pltpu.make_async_copy(
            tail_out,
            outT_hbm.at[:, pl.ds(_N_FULL * _CHUNK, _TAIL)],
            tail_sems.at[1],
        )

    @pl.when(i == 0)
    def _():
        in_copy(0, 0).start()

    @pl.when(i + 1 < _N_FULL)
    def _():
        in_copy(i + 1, nxt_slot).start()

    @pl.when(i + 1 == _N_FULL)
    def _():
        tail_in_copy().start()

    @pl.when(i < _N_FULL)
    def _():
        in_copy(i, slot).wait()

        @pl.when(i >= 2)
        def _():
            out_copy(slot, i - 2).wait()

        out_buf[slot] = (
            jnp.dot(wT_ref[...], in_buf[slot],
                    preferred_element_type=jnp.float32)
            + b_ref[...]
        )
        out_copy(slot, i).start()

    @pl.when(i == _N_FULL)
    def _():
        tail_in_copy().wait()
        tail_out[...] = (
            jnp.dot(wT_ref[...], tail_in[...],
                    preferred_element_type=jnp.float32)
            + b_ref[...]
        )
        tail_out_copy().start()
        out_copy(0, _N_FULL - 2).wait()
        out_copy(1, _N_FULL - 1).wait()
        tail_out_copy().wait()


def kernel(dummy, table, W, b):
    M, D = table.shape
    tT = table.T
    wT = W.T
    b_col = b.reshape(D, 1)

    outT = pl.pallas_call(
        _affine_kernel,
        grid=(_N_FULL + 1,),
        in_specs=[
            pl.BlockSpec((D, D), lambda i: (0, 0)),
            pl.BlockSpec((D, 1), lambda i: (0, 0)),
            pl.BlockSpec(memory_space=pltpu.MemorySpace.HBM),
        ],
        out_specs=pl.BlockSpec(memory_space=pltpu.MemorySpace.HBM),
        out_shape=jax.ShapeDtypeStruct((D, M), jnp.float32),
        scratch_shapes=[
            pltpu.VMEM((2, D, _CHUNK), jnp.float32),
            pltpu.VMEM((2, D, _CHUNK), jnp.float32),
            pltpu.VMEM((D, _TAIL), jnp.float32),
            pltpu.VMEM((D, _TAIL), jnp.float32),
            pltpu.SemaphoreType.DMA((2,)),
            pltpu.SemaphoreType.DMA((2,)),
            pltpu.SemaphoreType.DMA((2,)),
        ],
        compiler_params=pltpu.CompilerParams(
            dimension_semantics=("arbitrary",),
        ),
    )(wT, b_col, tT)
    return outT.T


# speedup vs baseline: 8.7468x; 1.0185x over previous
import jax
import jax.numpy as jnp
from jax.experimental import pallas as pl
from jax.experimental.pallas import tpu as pltpu

_CHUNK = 31232
_N_FULL = 32
_TAIL = 1_000_000 - _N_FULL * _CHUNK


def _matcol(w_ref, b_ref, x):
    prod = jax.lax.dot_general(
        w_ref[...], x, (((0,), (0,)), ((), ())),
        preferred_element_type=jnp.float32,
    )
    return prod + jnp.transpose(b_ref[...], (1, 0))


def _affine_kernel(w_ref, b_ref, tT_hbm, outT_hbm, in_buf, out_buf,
                   tail_in, tail_out, in_sems, out_sems, tail_sems):
    i = pl.program_id(0)
    slot = jax.lax.rem(i, 2)
    nxt_slot = jax.lax.rem(i + 1, 2)

    def in_copy(chunk_idx, buf_slot):
        return pltpu.make_async_copy(
            tT_hbm.at[:, pl.ds(chunk_idx * _CHUNK, _CHUNK)],
            in_buf.at[buf_slot],
            in_sems.at[buf_slot],
        )

    def out_copy(buf_slot, chunk_idx):
        return pltpu.make_async_copy(
            out_buf.at[buf_slot],
            outT_hbm.at[:, pl.ds(chunk_idx * _CHUNK, _CHUNK)],
            out_sems.at[buf_slot],
        )

    def tail_in_copy():
        return pltpu.make_async_copy(
            tT_hbm.at[:, pl.ds(_N_FULL * _CHUNK, _TAIL)],
            tail_in,
            tail_sems.at[0],
        )

    def tail_out_copy():
        return pltpu.make_async_copy(
            tail_out,
            outT_hbm.at[:, pl.ds(_N_FULL * _CHUNK, _TAIL)],
            tail_sems.at[1],
        )

    @pl.when(i == 0)
    def _():
        in_copy(0, 0).start()

    @pl.when(i + 1 < _N_FULL)
    def _():
        in_copy(i + 1, nxt_slot).start()

    @pl.when(i + 1 == _N_FULL)
    def _():
        tail_in_copy().start()

    @pl.when(i < _N_FULL)
    def _():
        in_copy(i, slot).wait()

        @pl.when(i >= 2)
        def _():
            out_copy(slot, i - 2).wait()

        out_buf[slot] = _matcol(w_ref, b_ref, in_buf[slot])
        out_copy(slot, i).start()

    @pl.when(i == _N_FULL)
    def _():
        tail_in_copy().wait()
        tail_out[...] = _matcol(w_ref, b_ref, tail_in[...])
        tail_out_copy().start()
        out_copy(0, _N_FULL - 2).wait()
        out_copy(1, _N_FULL - 1).wait()
        tail_out_copy().wait()


def kernel(dummy, table, W, b):
    M, D = table.shape
    tT = table.T
    b_row = b.reshape(1, D)

    outT = pl.pallas_call(
        _affine_kernel,
        grid=(_N_FULL + 1,),
        in_specs=[
            pl.BlockSpec((D, D), lambda i: (0, 0)),
            pl.BlockSpec((1, D), lambda i: (0, 0)),
            pl.BlockSpec(memory_space=pltpu.MemorySpace.HBM),
        ],
        out_specs=pl.BlockSpec(memory_space=pltpu.MemorySpace.HBM),
        out_shape=jax.ShapeDtypeStruct((D, M), jnp.float32),
        scratch_shapes=[
            pltpu.VMEM((2, D, _CHUNK), jnp.float32),
            pltpu.VMEM((2, D, _CHUNK), jnp.float32),
            pltpu.VMEM((D, _TAIL), jnp.float32),
            pltpu.VMEM((D, _TAIL), jnp.float32),
            pltpu.SemaphoreType.DMA((2,)),
            pltpu.SemaphoreType.DMA((2,)),
            pltpu.SemaphoreType.DMA((2,)),
        ],
        compiler_params=pltpu.CompilerParams(
            dimension_semantics=("arbitrary",),
        ),
    )(W, b_row, tT)
    return outT.T


# speedup vs baseline: 8.7973x; 1.0058x over previous
import jax
import jax.numpy as jnp
from jax.experimental import pallas as pl
from jax.experimental.pallas import tpu as pltpu

_SLOTS = 4
_CHUNK = 15616
_N_FULL = 64
_TAIL = 1_000_000 - _N_FULL * _CHUNK


def _matcol(w_ref, b_ref, x):
    prod = jax.lax.dot_general(
        w_ref[...], x, (((0,), (0,)), ((), ())),
        preferred_element_type=jnp.float32,
    )
    return prod + jnp.transpose(b_ref[...], (1, 0))


def _affine_kernel(w_ref, b_ref, tT_hbm, outT_hbm, in_buf, out_buf,
                   tail_in, tail_out, in_sems, out_sems, tail_sems):
    i = pl.program_id(0)
    slot = jax.lax.rem(i, _SLOTS)

    def in_copy(chunk_idx, buf_slot):
        return pltpu.make_async_copy(
            tT_hbm.at[:, pl.ds(chunk_idx * _CHUNK, _CHUNK)],
            in_buf.at[buf_slot],
            in_sems.at[buf_slot],
        )

    def out_copy(buf_slot, chunk_idx):
        return pltpu.make_async_copy(
            out_buf.at[buf_slot],
            outT_hbm.at[:, pl.ds(chunk_idx * _CHUNK, _CHUNK)],
            out_sems.at[buf_slot],
        )

    def tail_in_copy():
        return pltpu.make_async_copy(
            tT_hbm.at[:, pl.ds(_N_FULL * _CHUNK, _TAIL)],
            tail_in,
            tail_sems.at[0],
        )

    def tail_out_copy():
        return pltpu.make_async_copy(
            tail_out,
            outT_hbm.at[:, pl.ds(_N_FULL * _CHUNK, _TAIL)],
            tail_sems.at[1],
        )

    @pl.when(i == 0)
    def _():
        for k in range(min(_SLOTS - 1, _N_FULL)):
            in_copy(k, k % _SLOTS).start()

    pre = i + _SLOTS - 1

    @pl.when(pre < _N_FULL)
    def _():
        in_copy(pre, jax.lax.rem(pre, _SLOTS)).start()

    @pl.when(pre == _N_FULL)
    def _():
        tail_in_copy().start()

    @pl.when(i < _N_FULL)
    def _():
        in_copy(i, slot).wait()

        @pl.when(i >= _SLOTS)
        def _():
            out_copy(slot, i - _SLOTS).wait()

        out_buf[slot] = _matcol(w_ref, b_ref, in_buf[slot])
        out_copy(slot, i).start()

    @pl.when(i == _N_FULL)
    def _():
        tail_in_copy().wait()
        tail_out[...] = _matcol(w_ref, b_ref, tail_in[...])
        tail_out_copy().start()
        for k in range(_N_FULL - _SLOTS, _N_FULL):
            out_copy(k % _SLOTS, k).wait()
        tail_out_copy().wait()


def kernel(dummy, table, W, b):
    M, D = table.shape
    tT = table.T
    b_row = b.reshape(1, D)

    outT = pl.pallas_call(
        _affine_kernel,
        grid=(_N_FULL + 1,),
        in_specs=[
            pl.BlockSpec((D, D), lambda i: (0, 0)),
            pl.BlockSpec((1, D), lambda i: (0, 0)),
            pl.BlockSpec(memory_space=pltpu.MemorySpace.HBM),
        ],
        out_specs=pl.BlockSpec(memory_space=pltpu.MemorySpace.HBM),
        out_shape=jax.ShapeDtypeStruct((D, M), jnp.float32),
        scratch_shapes=[
            pltpu.VMEM((_SLOTS, D, _CHUNK), jnp.float32),
            pltpu.VMEM((_SLOTS, D, _CHUNK), jnp.float32),
            pltpu.VMEM((D, _TAIL), jnp.float32),
            pltpu.VMEM((D, _TAIL), jnp.float32),
            pltpu.SemaphoreType.DMA((_SLOTS,)),
            pltpu.SemaphoreType.DMA((_SLOTS,)),
            pltpu.SemaphoreType.DMA((2,)),
        ],
        compiler_params=pltpu.CompilerParams(
            dimension_semantics=("arbitrary",),
        ),
    )(W, b_row, tT)
    return outT.T
